# baseline (speedup 1.0000x reference)
import functools

import jax
import jax.numpy as jnp
from jax import lax
from jax.experimental import pallas as pl
from jax.experimental.pallas import tpu as pltpu
from jax.experimental.pallas import tpu_sc as plsc

N = 10000
E = 320000
D_IN = 128
D_HID = 64
NC = 2
NS = 16
NW = NC * NS
CH = 128
CPW = 80
NBUF = 2
NBLK = CPW // NBUF
EPAD = NW * CPW * CH
NPAD = 10240
H = NPAD // 2
RPT = NPAD // NS
RB2 = 1024


def _mm2_body(xlo_ref, xhi_ref, wl_ref, wr_ref, b_ref, y_ref, z_ref):
    xlo = xlo_ref[...]
    xhi = xhi_ref[...]
    wl = wl_ref[...]
    wr = wr_ref[...]
    b = b_ref[...]
    ylo = jnp.dot(xlo, wl, preferred_element_type=jnp.float32)
    yhi = jnp.dot(xhi, wl, preferred_element_type=jnp.float32)
    zlo = jnp.dot(xlo, wr, preferred_element_type=jnp.float32) + b
    zhi = jnp.dot(xhi, wr, preferred_element_type=jnp.float32) + b
    y_ref[...] = jnp.concatenate([ylo, yhi], axis=1)
    z_ref[...] = jnp.concatenate([zlo, zhi], axis=1)


def _phase_a(x, W1_l, W1_r, b1_2d):
    return pl.pallas_call(
        _mm2_body,
        grid=(H // RB2,),
        in_specs=[
            pl.BlockSpec((RB2, D_IN), lambda i: (i, 0)),
            pl.BlockSpec((RB2, D_IN), lambda i: (i + H // RB2, 0)),
            pl.BlockSpec((D_IN, D_HID), lambda i: (0, 0)),
            pl.BlockSpec((D_IN, D_HID), lambda i: (0, 0)),
            pl.BlockSpec((1, D_HID), lambda i: (0, 0)),
        ],
        out_specs=[
            pl.BlockSpec((RB2, 2 * D_HID), lambda i: (i, 0)),
            pl.BlockSpec((RB2, 2 * D_HID), lambda i: (i, 0)),
        ],
        out_shape=[
            jax.ShapeDtypeStruct((H, 2 * D_HID), jnp.float32),
            jax.ShapeDtypeStruct((H, 2 * D_HID), jnp.float32),
        ],
    )(x, x, W1_l, W1_r, b1_2d)


def _sc_agg64_body(ytab, srcm, dstm, agg_out, cnt_out,
                   idxs, idxd, rows, ones_v, zcnt, ytab_sh, agg_sh, cnt_sh,
                   gsem, ssem, csem):
    c = lax.axis_index("c")
    s = lax.axis_index("s")
    zero16 = jnp.zeros((16,), jnp.float32)
    one16 = jnp.ones((16,), jnp.float32)

    def zrow_body(i, carry):
        for j in range(D_HID // 16):
            rows[0, i, pl.ds(j * 16, 16)] = zero16
        return carry

    lax.fori_loop(0, CH, zrow_body, 0)

    def fill_body(i, carry):
        ones_v[pl.ds(i * 16, 16)] = one16
        return carry

    lax.fori_loop(0, CH // 16, fill_body, 0)

    def zcnt_body(i, carry):
        zcnt[pl.ds(i * 16, 16)] = zero16
        return carry

    lax.fori_loop(0, RPT // 16, zcnt_body, 0)

    base = (c * NS + s) * CPW
    pltpu.sync_copy(srcm.at[pl.ds(base, CPW)], idxs)
    pltpu.sync_copy(dstm.at[pl.ds(base, CPW)], idxd)

    pltpu.sync_copy(ytab.at[pl.ds(s * RPT, RPT)],
                    ytab_sh.at[pl.ds(s * RPT, RPT)])
    for k in range(RPT // CH):
        pltpu.sync_copy(rows.at[0], agg_sh.at[pl.ds(s * RPT + k * CH, CH)])
    pltpu.sync_copy(zcnt, cnt_sh.at[pl.ds(s * RPT, RPT)])
    plsc.subcore_barrier()

    for b in range(NBUF):
        pltpu.async_copy(ytab_sh.at[idxs.at[b]], rows.at[b], gsem.at[b])

    def blk(g, carry):
        for b in range(NBUF):
            i = g * NBUF + b
            pltpu.make_async_copy(ytab_sh.at[idxs.at[i]], rows.at[b],
                                  gsem.at[b]).wait()
            pltpu.async_copy(rows.at[b], agg_sh.at[idxd.at[i]], ssem.at[b],
                             add=True)
            pltpu.async_copy(ones_v, cnt_sh.at[idxd.at[i]], csem.at[b],
                             add=True)
        for b in range(NBUF):
            i = g * NBUF + b
            j = i + NBUF
            pltpu.make_async_copy(rows.at[b], agg_sh.at[idxd.at[i]],
                                  ssem.at[b]).wait()
            pltpu.make_async_copy(ones_v, cnt_sh.at[idxd.at[i]],
                                  csem.at[b]).wait()
            pltpu.async_copy(ytab_sh.at[idxs.at[j]], rows.at[b], gsem.at[b])
        return carry

    lax.fori_loop(0, NBLK - 1, blk, 0)

    g_last = NBLK - 1
    for b in range(NBUF):
        i = g_last * NBUF + b
        pltpu.make_async_copy(ytab_sh.at[idxs.at[i]], rows.at[b],
                              gsem.at[b]).wait()
        pltpu.async_copy(rows.at[b], agg_sh.at[idxd.at[i]], ssem.at[b],
                         add=True)
        pltpu.async_copy(ones_v, cnt_sh.at[idxd.at[i]], csem.at[b], add=True)
    for b in range(NBUF):
        i = g_last * NBUF + b
        pltpu.make_async_copy(rows.at[b], agg_sh.at[idxd.at[i]],
                              ssem.at[b]).wait()
        pltpu.make_async_copy(ones_v, cnt_sh.at[idxd.at[i]],
                              csem.at[b]).wait()
    plsc.subcore_barrier()

    pltpu.sync_copy(agg_sh.at[pl.ds(s * RPT, RPT)],
                    agg_out.at[c, pl.ds(s * RPT, RPT)])
    pltpu.sync_copy(cnt_sh.at[pl.ds(s * RPT, RPT)],
                    cnt_out.at[c, pl.ds(s * RPT, RPT)])


def _phase_b(ytab, srcm, dstm):
    mesh = plsc.VectorSubcoreMesh(core_axis_name="c", subcore_axis_name="s")
    f = functools.partial(
        pl.kernel,
        out_type=[
            jax.ShapeDtypeStruct((NC, NPAD, D_HID), jnp.float32),
            jax.ShapeDtypeStruct((NC, NPAD), jnp.float32),
        ],
        mesh=mesh,
        scratch_types=[
            pltpu.VMEM((CPW, CH), jnp.int32),
            pltpu.VMEM((CPW, CH), jnp.int32),
            pltpu.VMEM((NBUF, CH, D_HID), jnp.float32),
            pltpu.VMEM((CH,), jnp.float32),
            pltpu.VMEM((RPT,), jnp.float32),
            pltpu.VMEM_SHARED((NPAD, D_HID), jnp.float32),
            pltpu.VMEM_SHARED((NPAD, D_HID), jnp.float32),
            pltpu.VMEM_SHARED((NPAD,), jnp.float32),
            pltpu.SemaphoreType.DMA((NBUF,)),
            pltpu.SemaphoreType.DMA((NBUF,)),
            pltpu.SemaphoreType.DMA((NBUF,)),
        ],
        compiler_params=pltpu.CompilerParams(use_tc_tiling_on_sc=False),
    )(_sc_agg64_body)
    return f(ytab, srcm, dstm)


def _proj_body(aggp_ref, cnt_ref, z1_ref, wl_ref, wr_ref,
               y2_ref, z2_ref, rc_ref):
    ag = aggp_ref[0] + aggp_ref[1]
    alo = ag[:, :D_HID]
    ahi = ag[:, D_HID:]
    zlo = z1_ref[:, :D_HID]
    zhi = z1_ref[:, D_HID:]
    csum = jnp.maximum(cnt_ref[0] + cnt_ref[1], 1.0)
    rlo = 1.0 / csum[:, 0:1]
    rhi = 1.0 / csum[:, 1:2]
    hlo = jnp.maximum(alo * rlo + zlo, 0.0)
    hhi = jnp.maximum(ahi * rhi + zhi, 0.0)
    wl = wl_ref[...]
    wr = wr_ref[...]
    y2_ref[...] = jnp.concatenate(
        [jnp.dot(hlo, wl, preferred_element_type=jnp.float32),
         jnp.dot(hhi, wl, preferred_element_type=jnp.float32)], axis=1)
    z2_ref[...] = jnp.concatenate(
        [jnp.dot(hlo, wr, preferred_element_type=jnp.float32),
         jnp.dot(hhi, wr, preferred_element_type=jnp.float32)], axis=1)
    rc_ref[...] = jnp.concatenate([rlo, rhi], axis=1)


def _phase_c(aggv, cntp3, z1p, wl, wr):
    return pl.pallas_call(
        _proj_body,
        grid=(H // RB2,),
        in_specs=[
            pl.BlockSpec((NC, RB2, 2 * D_HID), lambda i: (0, i, 0)),
            pl.BlockSpec((NC, RB2, 2), lambda i: (0, i, 0)),
            pl.BlockSpec((RB2, 2 * D_HID), lambda i: (i, 0)),
            pl.BlockSpec((D_HID, 1), lambda i: (0, 0)),
            pl.BlockSpec((D_HID, 1), lambda i: (0, 0)),
        ],
        out_specs=[
            pl.BlockSpec((RB2, 2), lambda i: (i, 0)),
            pl.BlockSpec((RB2, 2), lambda i: (i, 0)),
            pl.BlockSpec((RB2, 2), lambda i: (i, 0)),
        ],
        out_shape=[
            jax.ShapeDtypeStruct((H, 2), jnp.float32),
            jax.ShapeDtypeStruct((H, 2), jnp.float32),
            jax.ShapeDtypeStruct((H, 2), jnp.float32),
        ],
    )(aggv, cntp3, z1p, wl, wr)


def _sc_agg1_body(y2v, srcm, dstm, agg_out,
                  idxs, idxd, vals, zcnt, y2_sh, agg2_sh, gsem, ssem):
    c = lax.axis_index("c")
    s = lax.axis_index("s")
    zero16 = jnp.zeros((16,), jnp.float32)

    def zcnt_body(i, carry):
        zcnt[pl.ds(i * 16, 16)] = zero16
        return carry

    lax.fori_loop(0, RPT // 16, zcnt_body, 0)

    base = (c * NS + s) * CPW
    pltpu.sync_copy(srcm.at[pl.ds(base, CPW)], idxs)
    pltpu.sync_copy(dstm.at[pl.ds(base, CPW)], idxd)

    pltpu.sync_copy(y2v.at[pl.ds(s * RPT, RPT)], y2_sh.at[pl.ds(s * RPT, RPT)])
    pltpu.sync_copy(zcnt, agg2_sh.at[pl.ds(s * RPT, RPT)])
    plsc.subcore_barrier()

    for b in range(NBUF):
        pltpu.async_copy(y2_sh.at[idxs.at[b]], vals.at[b], gsem.at[b])

    def blk(g, carry):
        for b in range(NBUF):
            i = g * NBUF + b
            pltpu.make_async_copy(y2_sh.at[idxs.at[i]], vals.at[b],
                                  gsem.at[b]).wait()
            pltpu.async_copy(vals.at[b], agg2_sh.at[idxd.at[i]], ssem.at[b],
                             add=True)
        for b in range(NBUF):
            i = g * NBUF + b
            pltpu.make_async_copy(vals.at[b], agg2_sh.at[idxd.at[i]],
                                  ssem.at[b]).wait()
            pltpu.async_copy(y2_sh.at[idxs.at[i + NBUF]], vals.at[b],
                             gsem.at[b])
        return carry

    lax.fori_loop(0, NBLK - 1, blk, 0)

    g_last = NBLK - 1
    for b in range(NBUF):
        i = g_last * NBUF + b
        pltpu.make_async_copy(y2_sh.at[idxs.at[i]], vals.at[b],
                              gsem.at[b]).wait()
        pltpu.async_copy(vals.at[b], agg2_sh.at[idxd.at[i]], ssem.at[b],
                         add=True)
    for b in range(NBUF):
        i = g_last * NBUF + b
        pltpu.make_async_copy(vals.at[b], agg2_sh.at[idxd.at[i]],
                              ssem.at[b]).wait()
    plsc.subcore_barrier()

    pltpu.sync_copy(agg2_sh.at[pl.ds(s * RPT, RPT)],
                    agg_out.at[c, pl.ds(s * RPT, RPT)])


def _phase_d(y2v, srcm, dstm):
    mesh = plsc.VectorSubcoreMesh(core_axis_name="c", subcore_axis_name="s")
    f = functools.partial(
        pl.kernel,
        out_type=jax.ShapeDtypeStruct((NC, NPAD), jnp.float32),
        mesh=mesh,
        scratch_types=[
            pltpu.VMEM((CPW, CH), jnp.int32),
            pltpu.VMEM((CPW, CH), jnp.int32),
            pltpu.VMEM((NBUF, CH), jnp.float32),
            pltpu.VMEM((RPT,), jnp.float32),
            pltpu.VMEM_SHARED((NPAD,), jnp.float32),
            pltpu.VMEM_SHARED((NPAD,), jnp.float32),
            pltpu.SemaphoreType.DMA((NBUF,)),
            pltpu.SemaphoreType.DMA((NBUF,)),
        ],
        compiler_params=pltpu.CompilerParams(use_tc_tiling_on_sc=False),
    )(_sc_agg1_body)
    return f(y2v, srcm, dstm)


NR = NPAD // 128


def _sig_body(a_ref, rc_ref, z2_ref, b2_ref, o_ref):
    t = ((a_ref[0] + a_ref[1]) * rc_ref[...] + z2_ref[...]
         + b2_ref[0, 0])
    o_ref[...] = 1.0 / (1.0 + jnp.exp(-t))


def _phase_e(agg2p, rcnt, z2, b2_2d):
    return pl.pallas_call(
        _sig_body,
        in_specs=[
            pl.BlockSpec((NC, NR, 128), lambda: (0, 0, 0)),
            pl.BlockSpec((NR, 128), lambda: (0, 0)),
            pl.BlockSpec((NR, 128), lambda: (0, 0)),
            pl.BlockSpec((1, 1), lambda: (0, 0)),
        ],
        out_specs=pl.BlockSpec((NR, 128), lambda: (0, 0)),
        out_shape=jax.ShapeDtypeStruct((NR, 128), jnp.float32),
    )(agg2p, rcnt, z2, b2_2d)


def kernel(x, edge_index, W1_l, b1, W1_r, W2_l, b2, W2_r):
    x = x.astype(jnp.float32)
    ei = edge_index.astype(jnp.int32)
    flat = jnp.arange(EPAD, dtype=jnp.int32)
    inb = flat < E
    src_n = jnp.where(inb, jnp.pad(ei[0], (0, EPAD - E)), flat % N)
    dst_n = jnp.where(inb, jnp.pad(ei[1], (0, EPAD - E)),
                      N + flat % (NPAD - N))
    src_k = jnp.where(src_n < H, 2 * src_n, 2 * src_n - (NPAD - 1))
    dst_k = jnp.where(dst_n < H, 2 * dst_n, 2 * dst_n - (NPAD - 1))
    srcm = src_k.reshape(NW * CPW, CH)
    dstm = dst_k.reshape(NW * CPW, CH)

    ytabp, z1p = _phase_a(x, W1_l, W1_r, b1[None, :])
    aggp, cntp = _phase_b(ytabp.reshape(NPAD, D_HID), srcm, dstm)
    y2p, z2p, rcp = _phase_c(aggp.reshape(NC, H, 2 * D_HID),
                             cntp.reshape(NC, H, 2), z1p, W2_l, W2_r)
    agg2p = _phase_d(y2p.reshape(NPAD), srcm, dstm)
    out = _phase_e(agg2p.reshape(NC, NR, 128), rcp.reshape(NR, 128),
                   z2p.reshape(NR, 128), b2[None, :])
    v = out.reshape(NPAD)
    return jnp.concatenate([v[0::2], v[1::2]])[:N].reshape(N, 1)
